# baseline (speedup 1.0000x reference)
import functools

import jax
import jax.numpy as jnp
from jax import lax
from jax.experimental import pallas as pl
from jax.experimental.pallas import tpu as pltpu
from jax.experimental.pallas import tpu_sc as plsc

BATCH = 128
SEQ = 512
D = 64
NC = 2
NS = 16
NW = NC * NS
RPW = BATCH * SEQ // NW
LANES = 16


def _sc_gather(idx2, table):
    mesh = plsc.VectorSubcoreMesh(core_axis_name="c", subcore_axis_name="s")

    @functools.partial(
        pl.kernel,
        out_type=jax.ShapeDtypeStruct((BATCH * SEQ, D), jnp.float32),
        mesh=mesh,
        scratch_types=[
            pltpu.VMEM((RPW,), jnp.int32),
            pltpu.SemaphoreType.DMA,
            pltpu.SemaphoreType.DMA,
        ],
        compiler_params=pltpu.CompilerParams(needs_layout_passes=False),
    )
    def k(idx_hbm, table_hbm, out_hbm, idx_v, isem, gsem):
        wid = lax.axis_index("s") * NC + lax.axis_index("c")
        base = wid * RPW
        pltpu.async_copy(idx_hbm.at[wid], idx_v, isem).wait()
        lane_iota = lax.iota(jnp.int32, LANES)

        @pl.loop(0, RPW // LANES)
        def _(c):
            v = idx_v[pl.ds(c * LANES, LANES)]
            for l in range(LANES):
                i = jnp.sum(jnp.where(lane_iota == l, v, 0))
                pltpu.async_copy(
                    table_hbm.at[i], out_hbm.at[base + c * LANES + l], gsem)

        pltpu.make_async_copy(
            table_hbm.at[pl.ds(0, RPW)],
            out_hbm.at[pl.ds(base, RPW)],
            gsem,
        ).wait()

    return k(idx2, table)


def _tc_add_relu(g, w_pos):
    BB = 8

    def body(g_ref, p_ref, o_ref):
        o_ref[...] = jnp.maximum(g_ref[...] + p_ref[...][None], 0.0)

    return pl.pallas_call(
        body,
        grid=(BATCH // BB,),
        in_specs=[
            pl.BlockSpec((BB, SEQ, D), lambda i: (i, 0, 0)),
            pl.BlockSpec((SEQ, D), lambda i: (0, 0)),
        ],
        out_specs=pl.BlockSpec((BB, SEQ, D), lambda i: (i, 0, 0)),
        out_shape=jax.ShapeDtypeStruct((BATCH, SEQ, D), jnp.float32),
    )(g, w_pos)


def kernel(X, W_word, W_pos):
    idx2 = X.astype(jnp.int32).reshape(NW, RPW)
    g = _sc_gather(idx2, W_word).reshape(BATCH, SEQ, D)
    return _tc_add_relu(g, W_pos)


# speedup vs baseline: 3.2798x; 3.2798x over previous
import functools

import jax
import jax.numpy as jnp
from jax import lax
from jax.experimental import pallas as pl
from jax.experimental.pallas import tpu as pltpu
from jax.experimental.pallas import tpu_sc as plsc

BATCH = 128
SEQ = 512
D = 64
NC = 2
NS = 16
NW = NC * NS
RPW = BATCH * SEQ // NW
CHUNK = 128
LANES = 16


def _sc_gather(idx2, table):
    mesh = plsc.VectorSubcoreMesh(core_axis_name="c", subcore_axis_name="s")

    @functools.partial(
        pl.kernel,
        out_type=jax.ShapeDtypeStruct((BATCH * SEQ, D), jnp.float32),
        mesh=mesh,
        scratch_types=[
            pltpu.VMEM((RPW,), jnp.int32),
            pltpu.VMEM((CHUNK, D), jnp.float32),
            pltpu.VMEM((CHUNK, D), jnp.float32),
            pltpu.SemaphoreType.DMA,
            pltpu.SemaphoreType.DMA,
            pltpu.SemaphoreType.DMA,
        ],
        compiler_params=pltpu.CompilerParams(needs_layout_passes=False),
    )
    def k(idx_hbm, table_hbm, out_hbm, idx_v, rows0, rows1, isem, gsem, osem):
        wid = lax.axis_index("s") * NC + lax.axis_index("c")
        base = wid * RPW
        pltpu.async_copy(idx_hbm.at[wid], idx_v, isem).wait()
        lane_iota = lax.iota(jnp.int32, LANES)

        def fire(c, buf):
            @pl.loop(0, CHUNK // LANES)
            def _(g):
                v = idx_v[pl.ds(c * CHUNK + g * LANES, LANES)]
                for l in range(LANES):
                    i = jnp.sum(jnp.where(lane_iota == l, v, 0))
                    pltpu.async_copy(table_hbm.at[i],
                                     buf.at[g * LANES + l], gsem)

        def drain_rows(buf):
            pltpu.make_async_copy(table_hbm.at[pl.ds(0, CHUNK)], buf, gsem
                                  ).wait()

        @pl.loop(0, RPW // (2 * CHUNK))
        def _(p):
            c0 = 2 * p
            fire(c0, rows0)
            fire(c0 + 1, rows1)
            drain_rows(rows0)
            pltpu.sync_copy(rows0, out_hbm.at[pl.ds(base + c0 * CHUNK, CHUNK)])
            drain_rows(rows1)
            pltpu.sync_copy(rows1,
                            out_hbm.at[pl.ds(base + (c0 + 1) * CHUNK, CHUNK)])

    return k(idx2, table)


def _tc_add_relu(g, w_pos):
    BB = 8

    def body(g_ref, p_ref, o_ref):
        o_ref[...] = jnp.maximum(g_ref[...] + p_ref[...][None], 0.0)

    return pl.pallas_call(
        body,
        grid=(BATCH // BB,),
        in_specs=[
            pl.BlockSpec((BB, SEQ, D), lambda i: (i, 0, 0)),
            pl.BlockSpec((SEQ, D), lambda i: (0, 0)),
        ],
        out_specs=pl.BlockSpec((BB, SEQ, D), lambda i: (i, 0, 0)),
        out_shape=jax.ShapeDtypeStruct((BATCH, SEQ, D), jnp.float32),
    )(g, w_pos)


def kernel(X, W_word, W_pos):
    idx2 = X.astype(jnp.int32).reshape(NW, RPW)
    g = _sc_gather(idx2, W_word).reshape(BATCH, SEQ, D)
    return _tc_add_relu(g, W_pos)
